# baseline (speedup 1.0000x reference)
import functools

import jax
import jax.numpy as jnp
from jax import lax
from jax.experimental import pallas as pl
from jax.experimental.pallas import tpu as pltpu
from jax.experimental.pallas import tpu_sc as plsc

N = 10000
D = 128
NC = 2
NS = 16
NW = NC * NS
CH = 128
N_PAD = 10112
ROWS_PER_TILE = N_PAD // NS

@functools.cache
def _mesh():
    return plsc.VectorSubcoreMesh(
        core_axis_name="c", subcore_axis_name="s",
        num_cores=NC, num_subcores=NS)


def _wid():
    return lax.axis_index("s") * NC + lax.axis_index("c")


def _deg_body(steps, dst_hbm, ones_hbm, zeros_hbm, out_hbm, dst_v, ones_v,
              sem, deg_sp):
    s = lax.axis_index("s")
    c = lax.axis_index("c")
    wid = s * NC + c
    pltpu.sync_copy(dst_hbm.at[wid], dst_v)
    pltpu.sync_copy(ones_hbm, ones_v)
    pltpu.sync_copy(zeros_hbm, deg_sp.at[pl.ds(s * ROWS_PER_TILE, ROWS_PER_TILE)])
    plsc.subcore_barrier()

    def step(j, carry):
        pltpu.sync_copy(ones_v, deg_sp.at[dst_v.at[j]], add=True)
        return carry

    lax.fori_loop(0, steps, step, 0)
    plsc.subcore_barrier()
    sl = pl.ds(s * ROWS_PER_TILE, ROWS_PER_TILE)
    pltpu.sync_copy(deg_sp.at[sl], out_hbm.at[c].at[sl])


def _make_deg(steps):
    return pl.kernel(
        functools.partial(_deg_body, steps),
        out_type=jax.ShapeDtypeStruct((NC, N_PAD, D), jnp.float32),
        mesh=_mesh(),
        scratch_types=[
            pltpu.VMEM((steps, CH), jnp.int32),
            pltpu.VMEM((CH, D), jnp.float32),
            pltpu.SemaphoreType.DMA,
            pltpu.VMEM_SHARED((N_PAD, D), jnp.float32),
        ],
    )


def _agg_body(steps, hs_hbm, src_hbm, dst_hbm, zeros_hbm, out_hbm,
              src_v, dst_v, rows_v, gsem, acc_sp):
    s = lax.axis_index("s")
    c = lax.axis_index("c")
    wid = s * NC + c
    pltpu.sync_copy(src_hbm.at[wid], src_v)
    pltpu.sync_copy(dst_hbm.at[wid], dst_v)
    pltpu.sync_copy(zeros_hbm, acc_sp.at[pl.ds(s * ROWS_PER_TILE, ROWS_PER_TILE)])
    plsc.subcore_barrier()

    def step(j, carry):
        pltpu.async_copy(hs_hbm.at[src_v.at[j]], rows_v, gsem).wait()
        pltpu.sync_copy(rows_v, acc_sp.at[dst_v.at[j]], add=True)
        return carry

    lax.fori_loop(0, steps, step, 0)
    plsc.subcore_barrier()
    sl = pl.ds(s * ROWS_PER_TILE, ROWS_PER_TILE)
    pltpu.sync_copy(acc_sp.at[sl], out_hbm.at[c].at[sl])


def _make_agg(steps):
    return pl.kernel(
        functools.partial(_agg_body, steps),
        out_type=jax.ShapeDtypeStruct((NC, N_PAD, D), jnp.float32),
        mesh=_mesh(),
        scratch_types=[
            pltpu.VMEM((steps, CH), jnp.int32),
            pltpu.VMEM((steps, CH), jnp.int32),
            pltpu.VMEM((CH, D), jnp.float32),
            pltpu.SemaphoreType.DMA,
            pltpu.VMEM_SHARED((N_PAD, D), jnp.float32),
        ],
    )


_BLK = 1000


def _dinv_of(deg_ref):
    deg = deg_ref[0, :, 0] + deg_ref[1, :, 0] + 1.0
    return lax.rsqrt(deg)[:, None]


def _tc0_body(x_ref, w_ref, h_ref):
    h_ref[...] = jnp.dot(
        x_ref[...], w_ref[...], preferred_element_type=jnp.float32)


def _tc1_body(h_ref, deg_ref, hs_ref):
    hs_ref[...] = h_ref[...] * _dinv_of(deg_ref)


def _tc2_body(p_ref, hs_ref, deg_ref, w_ref, b_ref, out_ref):
    dinv = _dinv_of(deg_ref)
    t = dinv * (p_ref[0] + p_ref[1] + hs_ref[...]) + b_ref[...]
    t = jnp.maximum(t, 0.0)
    out_ref[...] = jnp.dot(
        t, w_ref[...], preferred_element_type=jnp.float32
    ) * dinv


def _tc3_body(p_ref, hs_ref, deg_ref, b_ref, out_ref):
    dinv = _dinv_of(deg_ref)
    out_ref[...] = dinv * (p_ref[0] + p_ref[1] + hs_ref[...]) + b_ref[...]


_row_spec = pl.BlockSpec((_BLK, D), lambda i: (i, 0))
_full_w = pl.BlockSpec((D, D), lambda i: (0, 0))
_bias_spec = pl.BlockSpec((1, D), lambda i: (0, 0))
_deg_spec = pl.BlockSpec((NC, _BLK, D), lambda i: (0, i, 0))
_part_spec = pl.BlockSpec((NC, _BLK, D), lambda i: (0, i, 0))
_out_rows = jax.ShapeDtypeStruct((N, D), jnp.float32)

_tc0 = pl.pallas_call(
    _tc0_body, grid=(N // _BLK,),
    in_specs=[_row_spec, _full_w], out_specs=_row_spec,
    out_shape=_out_rows)
_tc1 = pl.pallas_call(
    _tc1_body, grid=(N // _BLK,),
    in_specs=[_row_spec, _deg_spec], out_specs=_row_spec,
    out_shape=_out_rows)
_tc2 = pl.pallas_call(
    _tc2_body, grid=(N // _BLK,),
    in_specs=[_part_spec, _row_spec, _deg_spec, _full_w, _bias_spec],
    out_specs=_row_spec, out_shape=_out_rows)
_tc3 = pl.pallas_call(
    _tc3_body, grid=(N // _BLK,),
    in_specs=[_part_spec, _row_spec, _deg_spec, _bias_spec],
    out_specs=_row_spec, out_shape=_out_rows)


@jax.jit
def kernel(x, edge_index, W0, b0, W1, b1):
    e = edge_index.shape[1]
    steps = -(-e // (NW * CH))
    steps = -(-steps // 4) * 4
    e_pad = steps * NW * CH

    src = jnp.concatenate(
        [edge_index[0], jnp.zeros((e_pad - e,), jnp.int32)]).reshape(NW, steps, CH)
    dst = jnp.concatenate(
        [edge_index[1], jnp.full((e_pad - e,), N, jnp.int32)]).reshape(NW, steps, CH)

    ones_rows = jnp.ones((CH, D), jnp.float32)
    zeros_acg = jnp.zeros((ROWS_PER_TILE, D), jnp.float32)
    b0r = b0.reshape(1, D)
    b1r = b1.reshape(1, D)

    deg_p = _make_deg(steps)(dst, ones_rows, zeros_acg)
    agg = _make_agg(steps)
    src4 = src
    dst4 = dst

    h0 = _tc0(x, W0)
    hs0 = _tc1(h0, deg_p)
    p0 = agg(hs0, src4, dst4, zeros_acg)
    hs1 = _tc2(p0, hs0, deg_p, W1, b0r)
    p1 = agg(hs1, src4, dst4, zeros_acg)
    return _tc3(p1, hs1, deg_p, b1r)


# speedup vs baseline: 1.5847x; 1.5847x over previous
import functools

import jax
import jax.numpy as jnp
from jax import lax
from jax.experimental import pallas as pl
from jax.experimental.pallas import tpu as pltpu
from jax.experimental.pallas import tpu_sc as plsc

N = 10000
D = 128
NC = 2
NS = 16
NW = NC * NS
CH = 128
N_PAD = 10240
ROWS_PER_TILE = N_PAD // NS

@functools.cache
def _mesh():
    return plsc.VectorSubcoreMesh(
        core_axis_name="c", subcore_axis_name="s",
        num_cores=NC, num_subcores=NS)


def _wid():
    return lax.axis_index("s") * NC + lax.axis_index("c")


def _deg_body(steps, dst_hbm, ones_hbm, zeros_hbm, out_hbm, dst_v, ones_v,
              sem, deg_sp):
    s = lax.axis_index("s")
    c = lax.axis_index("c")
    wid = s * NC + c
    pltpu.sync_copy(dst_hbm.at[wid], dst_v)
    pltpu.sync_copy(ones_hbm, ones_v)
    pltpu.sync_copy(zeros_hbm, deg_sp.at[pl.ds(s * ROWS_PER_TILE, ROWS_PER_TILE)])
    plsc.subcore_barrier()

    def step(j, carry):
        pltpu.sync_copy(ones_v, deg_sp.at[dst_v.at[j]], add=True)
        return carry

    lax.fori_loop(0, steps, step, 0)
    plsc.subcore_barrier()
    sl = pl.ds(s * ROWS_PER_TILE, ROWS_PER_TILE)
    pltpu.sync_copy(deg_sp.at[sl], out_hbm.at[c].at[sl])


def _make_deg(steps):
    return pl.kernel(
        functools.partial(_deg_body, steps),
        out_type=jax.ShapeDtypeStruct((NC, N_PAD, D), jnp.float32),
        mesh=_mesh(),
        scratch_types=[
            pltpu.VMEM((steps, CH), jnp.int32),
            pltpu.VMEM((CH, D), jnp.float32),
            pltpu.SemaphoreType.DMA,
            pltpu.VMEM_SHARED((N_PAD, D), jnp.float32),
        ],
    )


def _agg_body(steps, hs_hbm, src_hbm, dst_hbm, zeros_hbm, out_hbm,
              src_v, dst_v, rows_v, gsem, acc_sp):
    s = lax.axis_index("s")
    c = lax.axis_index("c")
    wid = s * NC + c
    pltpu.sync_copy(src_hbm.at[wid], src_v)
    pltpu.sync_copy(dst_hbm.at[wid], dst_v)
    pltpu.sync_copy(zeros_hbm, acc_sp.at[pl.ds(s * ROWS_PER_TILE, ROWS_PER_TILE)])
    plsc.subcore_barrier()

    def step(j, carry):
        pltpu.async_copy(hs_hbm.at[src_v.at[j]], rows_v, gsem).wait()
        pltpu.sync_copy(rows_v, acc_sp.at[dst_v.at[j]], add=True)
        return carry

    lax.fori_loop(0, steps, step, 0)
    plsc.subcore_barrier()
    sl = pl.ds(s * ROWS_PER_TILE, ROWS_PER_TILE)
    pltpu.sync_copy(acc_sp.at[sl], out_hbm.at[c].at[sl])


def _make_agg(steps):
    return pl.kernel(
        functools.partial(_agg_body, steps),
        out_type=jax.ShapeDtypeStruct((NC, N_PAD, D), jnp.float32),
        mesh=_mesh(),
        scratch_types=[
            pltpu.VMEM((steps, CH), jnp.int32),
            pltpu.VMEM((steps, CH), jnp.int32),
            pltpu.VMEM((CH, D), jnp.float32),
            pltpu.SemaphoreType.DMA,
            pltpu.VMEM_SHARED((N_PAD, D), jnp.float32),
        ],
    )


_BLK = 1000


def _dinv_of(deg_ref):
    deg = deg_ref[0, :, 0] + deg_ref[1, :, 0] + 1.0
    return lax.rsqrt(deg)[:, None]


def _tc1_body(x_ref, w_ref, deg_ref, hs_ref):
    hs_ref[...] = jnp.dot(
        x_ref[...], w_ref[...], preferred_element_type=jnp.float32
    ) * _dinv_of(deg_ref)


def _tc2_body(p_ref, hs_ref, deg_ref, w_ref, b_ref, out_ref):
    dinv = _dinv_of(deg_ref)
    t = dinv * (p_ref[0] + p_ref[1] + hs_ref[...]) + b_ref[...]
    t = jnp.maximum(t, 0.0)
    out_ref[...] = jnp.dot(
        t, w_ref[...], preferred_element_type=jnp.float32
    ) * dinv


def _tc3_body(p_ref, hs_ref, deg_ref, b_ref, out_ref):
    dinv = _dinv_of(deg_ref)
    out_ref[...] = dinv * (p_ref[0] + p_ref[1] + hs_ref[...]) + b_ref[...]


_row_spec = pl.BlockSpec((_BLK, D), lambda i: (i, 0))
_full_w = pl.BlockSpec((D, D), lambda i: (0, 0))
_bias_spec = pl.BlockSpec((1, D), lambda i: (0, 0))
_deg_spec = pl.BlockSpec((NC, _BLK, D), lambda i: (0, i, 0))
_part_spec = pl.BlockSpec((NC, _BLK, D), lambda i: (0, i, 0))
_out_rows = jax.ShapeDtypeStruct((N, D), jnp.float32)

_tc1 = pl.pallas_call(
    _tc1_body, grid=(N // _BLK,),
    in_specs=[_row_spec, _full_w, _deg_spec], out_specs=_row_spec,
    out_shape=_out_rows)
_tc2 = pl.pallas_call(
    _tc2_body, grid=(N // _BLK,),
    in_specs=[_part_spec, _row_spec, _deg_spec, _full_w, _bias_spec],
    out_specs=_row_spec, out_shape=_out_rows)
_tc3 = pl.pallas_call(
    _tc3_body, grid=(N // _BLK,),
    in_specs=[_part_spec, _row_spec, _deg_spec, _bias_spec],
    out_specs=_row_spec, out_shape=_out_rows)


@jax.jit
def kernel(x, edge_index, W0, b0, W1, b1):
    e = edge_index.shape[1]
    steps = -(-e // (NW * CH))
    e_pad = steps * NW * CH

    src = jnp.concatenate(
        [edge_index[0], jnp.zeros((e_pad - e,), jnp.int32)]).reshape(NW, steps, CH)
    dst = jnp.concatenate(
        [edge_index[1], jnp.full((e_pad - e,), N, jnp.int32)]).reshape(NW, steps, CH)

    ones_rows = jnp.ones((CH, D), jnp.float32)
    zeros_acg = jnp.zeros((ROWS_PER_TILE, D), jnp.float32)
    b0r = b0.reshape(1, D)
    b1r = b1.reshape(1, D)

    deg_p = _make_deg(steps)(dst, ones_rows, zeros_acg)
    agg = _make_agg(steps)
    src4 = src
    dst4 = dst

    hs0 = _tc1(x, W0, deg_p)
    p0 = agg(hs0, src4, dst4, zeros_acg)
    hs1 = _tc2(p0, hs0, deg_p, W1, b0r)
    p1 = agg(hs1, src4, dst4, zeros_acg)
    return _tc3(p1, hs1, deg_p, b1r)


# speedup vs baseline: 1.5970x; 1.0078x over previous
import functools

import jax
import jax.numpy as jnp
from jax import lax
from jax.experimental import pallas as pl
from jax.experimental.pallas import tpu as pltpu
from jax.experimental.pallas import tpu_sc as plsc

N = 10000
D = 128
NC = 2
NS = 16
NW = NC * NS
CH = 128
N_PAD = 10240
ROWS_PER_TILE = N_PAD // NS

@functools.cache
def _mesh():
    return plsc.VectorSubcoreMesh(
        core_axis_name="c", subcore_axis_name="s",
        num_cores=NC, num_subcores=NS)


def _wid():
    return lax.axis_index("s") * NC + lax.axis_index("c")


def _deg_body(steps, dst_hbm, ones_hbm, zeros_hbm, out_hbm, dst_v, ones_v,
              sem, deg_sp):
    s = lax.axis_index("s")
    c = lax.axis_index("c")
    wid = s * NC + c
    pltpu.sync_copy(dst_hbm.at[wid], dst_v)
    pltpu.sync_copy(ones_hbm, ones_v)
    pltpu.sync_copy(zeros_hbm, deg_sp.at[pl.ds(s * ROWS_PER_TILE, ROWS_PER_TILE)])
    plsc.subcore_barrier()

    def step(j, carry):
        pltpu.sync_copy(ones_v, deg_sp.at[dst_v.at[j]], add=True)
        return carry

    lax.fori_loop(0, steps, step, 0)
    plsc.subcore_barrier()
    sl = pl.ds(s * ROWS_PER_TILE, ROWS_PER_TILE)
    pltpu.sync_copy(deg_sp.at[sl], out_hbm.at[c].at[sl])


def _make_deg(steps):
    return pl.kernel(
        functools.partial(_deg_body, steps),
        out_type=jax.ShapeDtypeStruct((NC, N_PAD, D), jnp.float32),
        mesh=_mesh(),
        scratch_types=[
            pltpu.VMEM((steps, CH), jnp.int32),
            pltpu.VMEM((CH, D), jnp.float32),
            pltpu.SemaphoreType.DMA,
            pltpu.VMEM_SHARED((N_PAD, D), jnp.float32),
        ],
    )


def _agg_body(steps, hs_hbm, src_hbm, dst_hbm, zeros_hbm, out_hbm,
              src_v, dst_v, rows_v, gsem, acc_sp):
    s = lax.axis_index("s")
    c = lax.axis_index("c")
    wid = s * NC + c
    pltpu.sync_copy(src_hbm.at[wid], src_v)
    pltpu.sync_copy(dst_hbm.at[wid], dst_v)
    pltpu.sync_copy(zeros_hbm, acc_sp.at[pl.ds(s * ROWS_PER_TILE, ROWS_PER_TILE)])
    plsc.subcore_barrier()

    def step(j, carry):
        pltpu.async_copy(hs_hbm.at[src_v.at[j]], rows_v, gsem).wait()
        pltpu.sync_copy(rows_v, acc_sp.at[dst_v.at[j]], add=True)
        return carry

    lax.fori_loop(0, steps, step, 0)
    plsc.subcore_barrier()
    sl = pl.ds(s * ROWS_PER_TILE, ROWS_PER_TILE)
    pltpu.sync_copy(acc_sp.at[sl], out_hbm.at[c].at[sl])


def _make_agg(steps):
    return pl.kernel(
        functools.partial(_agg_body, steps),
        out_type=jax.ShapeDtypeStruct((NC, N_PAD, D), jnp.float32),
        mesh=_mesh(),
        scratch_types=[
            pltpu.VMEM((steps, CH), jnp.int32),
            pltpu.VMEM((steps, CH), jnp.int32),
            pltpu.VMEM((CH, D), jnp.float32),
            pltpu.SemaphoreType.DMA,
            pltpu.VMEM_SHARED((N_PAD, D), jnp.float32),
        ],
    )


_BLK = 1000


def _dinv_of(deg_ref):
    deg = deg_ref[0, :, 0] + deg_ref[1, :, 0] + 1.0
    return lax.rsqrt(deg)[:, None]


def _tc1_body(x_ref, w_ref, deg_ref, hs_ref):
    hs_ref[...] = jnp.dot(
        x_ref[...], w_ref[...], preferred_element_type=jnp.float32
    ) * _dinv_of(deg_ref)


def _tc2_body(p_ref, hs_ref, deg_ref, w_ref, b_ref, out_ref):
    dinv = _dinv_of(deg_ref)
    t = dinv * (p_ref[0] + p_ref[1] + hs_ref[...]) + b_ref[...]
    t = jnp.maximum(t, 0.0)
    out_ref[...] = jnp.dot(
        t, w_ref[...], preferred_element_type=jnp.float32
    ) * dinv


def _tc3_body(p_ref, hs_ref, deg_ref, b_ref, out_ref):
    dinv = _dinv_of(deg_ref)
    out_ref[...] = dinv * (p_ref[0] + p_ref[1] + hs_ref[...]) + b_ref[...]


_row_spec = pl.BlockSpec((_BLK, D), lambda i: (i, 0))
_full_w = pl.BlockSpec((D, D), lambda i: (0, 0))
_bias_spec = pl.BlockSpec((1, D), lambda i: (0, 0))
_deg_spec = pl.BlockSpec((NC, _BLK, D), lambda i: (0, i, 0))
_part_spec = pl.BlockSpec((NC, _BLK, D), lambda i: (0, i, 0))
_out_rows = jax.ShapeDtypeStruct((N, D), jnp.float32)

_tc1 = pl.pallas_call(
    _tc1_body, grid=(N // _BLK,),
    in_specs=[_row_spec, _full_w, _deg_spec], out_specs=_row_spec,
    out_shape=_out_rows)
_tc2 = pl.pallas_call(
    _tc2_body, grid=(N // _BLK,),
    in_specs=[_part_spec, _row_spec, _deg_spec, _full_w, _bias_spec],
    out_specs=_row_spec, out_shape=_out_rows)
_tc3 = pl.pallas_call(
    _tc3_body, grid=(N // _BLK,),
    in_specs=[_part_spec, _row_spec, _deg_spec, _bias_spec],
    out_specs=_row_spec, out_shape=_out_rows)


@jax.jit
def kernel(x, edge_index, W0, b0, W1, b1):
    e = edge_index.shape[1]
    steps = -(-e // (NW * CH))
    e_pad = steps * NW * CH

    src = jnp.concatenate(
        [edge_index[0], jnp.zeros((e_pad - e,), jnp.int32)]).reshape(NW, steps, CH)
    trash = N + jnp.arange(e_pad - e, dtype=jnp.int32) % (N_PAD - N)
    dst = jnp.concatenate([edge_index[1], trash]).reshape(NW, steps, CH)

    ones_rows = jnp.ones((CH, D), jnp.float32)
    zeros_acg = jnp.zeros((ROWS_PER_TILE, D), jnp.float32)
    b0r = b0.reshape(1, D)
    b1r = b1.reshape(1, D)

    deg_p = _make_deg(steps)(dst, ones_rows, zeros_acg)
    agg = _make_agg(steps)
    src4 = src
    dst4 = dst

    hs0 = _tc1(x, W0, deg_p)
    p0 = agg(hs0, src4, dst4, zeros_acg)
    hs1 = _tc2(p0, hs0, deg_p, W1, b0r)
    p1 = agg(hs1, src4, dst4, zeros_acg)
    return _tc3(p1, hs1, deg_p, b1r)
